# Initial kernel scaffold; baseline (speedup 1.0000x reference)
import jax
import jax.numpy as jnp
from jax.experimental import pallas as pl


def kernel(x, edge_index, edge_attr, W1, b1, W2, b2):
    raise NotImplementedError("write your pallas kernel here")


# speedup vs baseline: 6.8124x; 6.8124x over previous
import functools

import jax
import jax.numpy as jnp
from jax import lax
from jax.experimental import pallas as pl
from jax.experimental.pallas import tpu as pltpu
from jax.experimental.pallas import tpu_sc as plsc

N = 10000
NP = 10240
D = 128
EP = 327680
CH = 128
NC, NS = 2, 16
NW = NC * NS
CPW = EP // CH // NW
RPT = NP // NS
LANES = 16

_mesh = plsc.VectorSubcoreMesh(
    core_axis_name="c", subcore_axis_name="s", num_cores=NC, num_subcores=NS
)


def _zeros16():
    return jnp.zeros((LANES,), jnp.float32)


def _sc_deg_body(dst_hbm, ew_hbm, out_hbm, dst_v, ew_v, deg_sp, sem):
    cid = lax.axis_index("c")
    sid = lax.axis_index("s")
    wid = sid * NC + cid

    for i in range(CH // LANES):
        ew_v[pl.ds(i * LANES, LANES)] = _zeros16()
    for i in range(RPT // CH):
        pltpu.sync_copy(ew_v, deg_sp.at[pl.ds(sid * RPT + i * CH, CH)])
    plsc.subcore_barrier()

    def body(i, carry):
        t = wid * CPW + i
        pltpu.sync_copy(dst_hbm.at[pl.ds(t * CH, CH)], dst_v)
        pltpu.sync_copy(ew_hbm.at[pl.ds(t * CH, CH)], ew_v)
        pltpu.sync_copy(ew_v, deg_sp.at[dst_v], add=True)
        return carry

    lax.fori_loop(0, CPW, body, 0)
    plsc.subcore_barrier()

    def wr(i, carry):
        off = sid * RPT + i * CH
        pltpu.sync_copy(deg_sp.at[pl.ds(off, CH)], ew_v)
        pltpu.sync_copy(ew_v, out_hbm.at[cid, pl.ds(off, CH)])
        return carry

    lax.fori_loop(0, RPT // CH, wr, 0)


_sc_deg = pl.kernel(
    _sc_deg_body,
    out_type=jax.ShapeDtypeStruct((NC, NP), jnp.float32),
    mesh=_mesh,
    scratch_types=[
        pltpu.VMEM((CH,), jnp.int32),
        pltpu.VMEM((CH,), jnp.float32),
        pltpu.VMEM_SHARED((NP,), jnp.float32),
        pltpu.SemaphoreType.DMA,
    ],
)


def _sc_agg_body(src_hbm, dst_hbm, ew_hbm, g_hbm, out_hbm,
                 src_v, dst_v, ew_v, rows_v, agg_sp, sem):
    cid = lax.axis_index("c")
    sid = lax.axis_index("s")
    wid = sid * NC + cid

    def zb(i, carry):
        for k in range(D // LANES):
            rows_v[i, pl.ds(k * LANES, LANES)] = _zeros16()
        return carry

    lax.fori_loop(0, CH, zb, 0)

    def zs(i, carry):
        pltpu.sync_copy(rows_v, agg_sp.at[pl.ds(sid * RPT + i * CH, CH)])
        return carry

    lax.fori_loop(0, RPT // CH, zs, 0)
    plsc.subcore_barrier()

    def body(i, carry):
        t = wid * CPW + i
        pltpu.sync_copy(src_hbm.at[pl.ds(t * CH, CH)], src_v)
        pltpu.sync_copy(dst_hbm.at[pl.ds(t * CH, CH)], dst_v)
        pltpu.sync_copy(ew_hbm.at[pl.ds(t * CH, CH)], ew_v)
        pltpu.async_copy(g_hbm.at[src_v], rows_v, sem).wait()

        def scale(gi, c2):
            e0 = gi * LANES
            ewg = ew_v[pl.ds(e0, LANES)]
            for j in range(LANES):
                w = jnp.broadcast_to(ewg[j], (LANES,))
                for k in range(D // LANES):
                    sl = pl.ds(k * LANES, LANES)
                    rows_v[e0 + j, sl] = rows_v[e0 + j, sl] * w
            return c2

        lax.fori_loop(0, CH // LANES, scale, 0)
        pltpu.sync_copy(rows_v, agg_sp.at[dst_v], add=True)
        return carry

    lax.fori_loop(0, CPW, body, 0)
    plsc.subcore_barrier()

    def wr(i, carry):
        off = sid * RPT + i * CH
        pltpu.sync_copy(agg_sp.at[pl.ds(off, CH)], rows_v)
        pltpu.sync_copy(rows_v, out_hbm.at[cid, pl.ds(off, CH)])
        return carry

    lax.fori_loop(0, RPT // CH, wr, 0)


_sc_agg = pl.kernel(
    _sc_agg_body,
    out_type=jax.ShapeDtypeStruct((NC, NP, D), jnp.float32),
    mesh=_mesh,
    scratch_types=[
        pltpu.VMEM((CH,), jnp.int32),
        pltpu.VMEM((CH,), jnp.int32),
        pltpu.VMEM((CH,), jnp.float32),
        pltpu.VMEM((CH, D), jnp.float32),
        pltpu.VMEM_SHARED((NP, D), jnp.float32),
        pltpu.SemaphoreType.DMA,
    ],
)


def _tc_dinv_body(degp_ref, out_ref):
    deg = 1.0 + degp_ref[0] + degp_ref[1]
    out_ref[...] = lax.rsqrt(deg)


def _tc_dinv(degp):
    return pl.pallas_call(
        _tc_dinv_body,
        out_shape=jax.ShapeDtypeStruct((NP // D, D), jnp.float32),
    )(degp.reshape(NC, NP // D, D))


RB = 1024
GRID = NP // RB


def _tc_mm_scale_body(x_ref, w_ref, dinv_ref, out_ref):
    acc = jnp.dot(x_ref[...], w_ref[...], preferred_element_type=jnp.float32)
    out_ref[...] = acc * dinv_ref[...]


def _tc_mm_scale(x, w, dinv_col):
    return pl.pallas_call(
        _tc_mm_scale_body,
        grid=(GRID,),
        in_specs=[
            pl.BlockSpec((RB, D), lambda i: (i, 0)),
            pl.BlockSpec((D, D), lambda i: (0, 0)),
            pl.BlockSpec((RB, 1), lambda i: (i, 0)),
        ],
        out_specs=pl.BlockSpec((RB, D), lambda i: (i, 0)),
        out_shape=jax.ShapeDtypeStruct((NP, D), jnp.float32),
    )(x, w, dinv_col)


def _tc_mid_body(aggp_ref, g_ref, dinv_ref, b_ref, w_ref, out_ref):
    agg = aggp_ref[0] + aggp_ref[1]
    h = jnp.maximum(dinv_ref[...] * (agg + g_ref[...]) + b_ref[...], 0.0)
    acc = jnp.dot(h, w_ref[...], preferred_element_type=jnp.float32)
    out_ref[...] = acc * dinv_ref[...]


def _tc_mid(aggp, g, dinv_col, b_row, w):
    return pl.pallas_call(
        _tc_mid_body,
        grid=(GRID,),
        in_specs=[
            pl.BlockSpec((NC, RB, D), lambda i: (0, i, 0)),
            pl.BlockSpec((RB, D), lambda i: (i, 0)),
            pl.BlockSpec((RB, 1), lambda i: (i, 0)),
            pl.BlockSpec((1, D), lambda i: (0, 0)),
            pl.BlockSpec((D, D), lambda i: (0, 0)),
        ],
        out_specs=pl.BlockSpec((RB, D), lambda i: (i, 0)),
        out_shape=jax.ShapeDtypeStruct((NP, D), jnp.float32),
    )(aggp, g, dinv_col, b_row, w)


def _tc_final_body(aggp_ref, g_ref, dinv_ref, b_ref, out_ref):
    agg = aggp_ref[0] + aggp_ref[1]
    out_ref[...] = dinv_ref[...] * (agg + g_ref[...]) + b_ref[...]


def _tc_final(aggp, g, dinv_col, b_row):
    return pl.pallas_call(
        _tc_final_body,
        grid=(GRID,),
        in_specs=[
            pl.BlockSpec((NC, RB, D), lambda i: (0, i, 0)),
            pl.BlockSpec((RB, D), lambda i: (i, 0)),
            pl.BlockSpec((RB, 1), lambda i: (i, 0)),
            pl.BlockSpec((1, D), lambda i: (0, 0)),
        ],
        out_specs=pl.BlockSpec((RB, D), lambda i: (i, 0)),
        out_shape=jax.ShapeDtypeStruct((NP, D), jnp.float32),
    )(aggp, g, dinv_col, b_row)


@jax.jit
def kernel(x, edge_index, edge_attr, W1, b1, W2, b2):
    E = edge_index.shape[1]
    src = edge_index[0].astype(jnp.int32)
    dst = edge_index[1].astype(jnp.int32)
    ew = edge_attr[:, 0].astype(jnp.float32)

    src_p = jnp.pad(src, (0, EP - E))
    dst_p = jnp.pad(dst, (0, EP - E))
    ew_p = jnp.pad(ew, (0, EP - E))
    x_p = jnp.pad(x, ((0, NP - N), (0, 0)))

    degp = _sc_deg(dst_p, ew_p)
    dinv_col = _tc_dinv(degp).reshape(NP, 1)

    g1 = _tc_mm_scale(x_p, W1, dinv_col)
    agg1 = _sc_agg(src_p, dst_p, ew_p, g1)
    g2 = _tc_mid(agg1, g1, dinv_col, b1.reshape(1, D), W2)
    agg2 = _sc_agg(src_p, dst_p, ew_p, g2)
    out = _tc_final(agg2, g2, dinv_col, b2.reshape(1, D))
    return out[:N]


# speedup vs baseline: 9.1980x; 1.3502x over previous
import functools

import jax
import jax.numpy as jnp
from jax import lax
from jax.experimental import pallas as pl
from jax.experimental.pallas import tpu as pltpu
from jax.experimental.pallas import tpu_sc as plsc

N = 10000
NP = 10240
D = 128
EP = 327680
CH = 128
NC, NS = 2, 16
NW = NC * NS
NCHUNK = EP // CH
CPW = NCHUNK // NW
RPT = NP // NS
LANES = 16

_mesh = plsc.VectorSubcoreMesh(
    core_axis_name="c", subcore_axis_name="s", num_cores=NC, num_subcores=NS
)


def _zeros16():
    return jnp.zeros((LANES,), jnp.float32)


def _sc_deg_body(dst_hbm, ew_hbm, out_hbm, dst_all, ew_all, buf_v, deg_sp,
                 sem_i, sem_s):
    cid = lax.axis_index("c")
    sid = lax.axis_index("s")
    wid = sid * NC + cid

    ld_d = pltpu.async_copy(dst_hbm.at[pl.ds(wid * CPW, CPW)], dst_all, sem_i)
    ld_e = pltpu.async_copy(ew_hbm.at[pl.ds(wid * CPW, CPW)], ew_all, sem_i)

    for i in range(CH // LANES):
        buf_v[pl.ds(i * LANES, LANES)] = _zeros16()
    for i in range(RPT // CH):
        pltpu.sync_copy(buf_v, deg_sp.at[pl.ds(sid * RPT + i * CH, CH)])
    ld_d.wait()
    ld_e.wait()
    plsc.subcore_barrier()

    K = 8

    def fire(c):
        pltpu.async_copy(ew_all.at[c], deg_sp.at[dst_all.at[c]], sem_s,
                         add=True)

    def drain(c):
        pltpu.make_async_copy(ew_all.at[c], deg_sp.at[dst_all.at[c]],
                              sem_s).wait()

    def body(i, carry):
        fire(K + i)
        drain(i)
        return carry

    for c in range(K):
        fire(c)
    lax.fori_loop(0, CPW - K, body, 0)
    for c in range(K):
        drain(CPW - K + c)
    plsc.subcore_barrier()

    def wr(i, carry):
        off = sid * RPT + i * CH
        pltpu.sync_copy(deg_sp.at[pl.ds(off, CH)], buf_v)
        pltpu.sync_copy(buf_v, out_hbm.at[cid, pl.ds(off, CH)])
        return carry

    lax.fori_loop(0, RPT // CH, wr, 0)


_sc_deg = pl.kernel(
    _sc_deg_body,
    out_type=jax.ShapeDtypeStruct((NC, NP), jnp.float32),
    mesh=_mesh,
    scratch_types=[
        pltpu.VMEM((CPW, CH), jnp.int32),
        pltpu.VMEM((CPW, CH), jnp.float32),
        pltpu.VMEM((CH,), jnp.float32),
        pltpu.VMEM_SHARED((NP,), jnp.float32),
        pltpu.SemaphoreType.DMA,
        pltpu.SemaphoreType.DMA,
    ],
)


PB = 8
NB = CPW // PB


def _sc_agg_body(src_hbm, dst_hbm, ew_hbm, g_hbm, out_hbm,
                 srcA, dstA, ewA, srcB, dstB, ewB, buf0, buf1, agg_sp,
                 sem_i, g0, g1, s0, s1):
    cid = lax.axis_index("c")
    sid = lax.axis_index("s")
    wid = sid * NC + cid

    def idx_refill(bidx, S, Dd, Ew):
        off = wid * CPW + bidx * PB
        pltpu.async_copy(src_hbm.at[pl.ds(off, PB)], S, sem_i)
        pltpu.async_copy(dst_hbm.at[pl.ds(off, PB)], Dd, sem_i)
        pltpu.async_copy(ew_hbm.at[pl.ds(off, PB)], Ew, sem_i)

    def idx_wait(bidx, S, Dd, Ew):
        off = wid * CPW + bidx * PB
        pltpu.make_async_copy(src_hbm.at[pl.ds(off, PB)], S, sem_i).wait()
        pltpu.make_async_copy(dst_hbm.at[pl.ds(off, PB)], Dd, sem_i).wait()
        pltpu.make_async_copy(ew_hbm.at[pl.ds(off, PB)], Ew, sem_i).wait()

    idx_refill(0, srcA, dstA, ewA)
    idx_refill(1, srcB, dstB, ewB)

    def zb(i, carry):
        for k in range(D // LANES):
            buf0[i, pl.ds(k * LANES, LANES)] = _zeros16()
        return carry

    lax.fori_loop(0, CH, zb, 0)

    def zs(i, carry):
        pltpu.sync_copy(buf0, agg_sp.at[pl.ds(sid * RPT + i * CH, CH)])
        return carry

    lax.fori_loop(0, RPT // CH, zs, 0)
    idx_wait(0, srcA, dstA, ewA)
    plsc.subcore_barrier()

    def start_gather(idx_ref, buf, sem):
        pltpu.async_copy(g_hbm.at[idx_ref], buf, sem)

    def wait_gather(idx_ref, buf, sem):
        pltpu.make_async_copy(g_hbm.at[idx_ref], buf, sem).wait()

    def start_scatter(idx_ref, buf, sem):
        pltpu.async_copy(buf, agg_sp.at[idx_ref], sem, add=True)

    def wait_scatter(idx_ref, buf, sem):
        pltpu.make_async_copy(buf, agg_sp.at[idx_ref], sem).wait()

    def scale(ew_row, buf):
        def grp(gi, carry):
            e0 = gi * LANES
            ewg = ew_row[pl.ds(e0, LANES)]
            for j in range(LANES):
                w = jnp.broadcast_to(ewg[j], (LANES,))
                for k in range(D // LANES):
                    sl = pl.ds(k * LANES, LANES)
                    buf[e0 + j, sl] = buf[e0 + j, sl] * w
            return carry

        lax.fori_loop(0, CH // LANES, grp, 0)

    start_gather(srcA.at[0], buf0, g0)
    start_gather(srcA.at[1], buf1, g1)

    def section(b, S, Dd, Ew, S2, D2, E2):
        def half(i, r, nxt, buf, gsem, ssem):
            wait_gather(S.at[r], buf, gsem)
            scale(Ew.at[r], buf)
            start_scatter(Dd.at[r], buf, ssem)
            wait_scatter(Dd.at[r], buf, ssem)

            @pl.when(i < PB // 2 - 1)
            def _():
                start_gather(S.at[r + 2], buf, gsem)

            @pl.when((i == PB // 2 - 1) & (b + 1 < NB))
            def _():
                if nxt == 0:
                    idx_wait(b + 1, S2, D2, E2)
                start_gather(S2.at[nxt], buf, gsem)

        def pair(i, carry):
            half(i, 2 * i, 0, buf0, g0, s0)
            half(i, 2 * i + 1, 1, buf1, g1, s1)
            return carry

        lax.fori_loop(0, PB // 2, pair, 0)

    def outer(ob, carry):
        bA = 2 * ob
        bB = 2 * ob + 1

        section(bA, srcA, dstA, ewA, srcB, dstB, ewB)

        @pl.when(bA + 2 < NB)
        def _():
            idx_refill(bA + 2, srcA, dstA, ewA)

        section(bB, srcB, dstB, ewB, srcA, dstA, ewA)

        @pl.when(bB + 2 < NB)
        def _():
            idx_refill(bB + 2, srcB, dstB, ewB)

        return carry

    lax.fori_loop(0, NB // 2, outer, 0)
    plsc.subcore_barrier()

    def wr(i, carry):
        off = sid * RPT + i * CH
        pltpu.sync_copy(agg_sp.at[pl.ds(off, CH)], buf0)
        pltpu.sync_copy(buf0, out_hbm.at[cid, pl.ds(off, CH)])
        return carry

    lax.fori_loop(0, RPT // CH, wr, 0)


_sc_agg = pl.kernel(
    _sc_agg_body,
    out_type=jax.ShapeDtypeStruct((NC, NP, D), jnp.float32),
    mesh=_mesh,
    scratch_types=[
        pltpu.VMEM((PB, CH), jnp.int32),
        pltpu.VMEM((PB, CH), jnp.int32),
        pltpu.VMEM((PB, CH), jnp.float32),
        pltpu.VMEM((PB, CH), jnp.int32),
        pltpu.VMEM((PB, CH), jnp.int32),
        pltpu.VMEM((PB, CH), jnp.float32),
        pltpu.VMEM((CH, D), jnp.float32),
        pltpu.VMEM((CH, D), jnp.float32),
        pltpu.VMEM_SHARED((NP, D), jnp.float32),
        pltpu.SemaphoreType.DMA,
        pltpu.SemaphoreType.DMA,
        pltpu.SemaphoreType.DMA,
        pltpu.SemaphoreType.DMA,
        pltpu.SemaphoreType.DMA,
    ],
)


def _tc_dinv_body(degp_ref, out_ref):
    deg = 1.0 + degp_ref[0] + degp_ref[1]
    out_ref[...] = lax.rsqrt(deg)


def _tc_dinv(degp):
    return pl.pallas_call(
        _tc_dinv_body,
        out_shape=jax.ShapeDtypeStruct((NP // D, D), jnp.float32),
    )(degp.reshape(NC, NP // D, D))


RB = 1024
GRID = NP // RB


def _tc_mm_scale_body(x_ref, w_ref, dinv_ref, out_ref):
    acc = jnp.dot(x_ref[...], w_ref[...], preferred_element_type=jnp.float32)
    out_ref[...] = acc * dinv_ref[...]


def _tc_mm_scale(x, w, dinv_col):
    return pl.pallas_call(
        _tc_mm_scale_body,
        grid=(GRID,),
        in_specs=[
            pl.BlockSpec((RB, D), lambda i: (i, 0)),
            pl.BlockSpec((D, D), lambda i: (0, 0)),
            pl.BlockSpec((RB, 1), lambda i: (i, 0)),
        ],
        out_specs=pl.BlockSpec((RB, D), lambda i: (i, 0)),
        out_shape=jax.ShapeDtypeStruct((NP, D), jnp.float32),
    )(x, w, dinv_col)


def _tc_mid_body(aggp_ref, g_ref, dinv_ref, b_ref, w_ref, out_ref):
    agg = aggp_ref[0] + aggp_ref[1]
    h = jnp.maximum(dinv_ref[...] * (agg + g_ref[...]) + b_ref[...], 0.0)
    acc = jnp.dot(h, w_ref[...], preferred_element_type=jnp.float32)
    out_ref[...] = acc * dinv_ref[...]


def _tc_mid(aggp, g, dinv_col, b_row, w):
    return pl.pallas_call(
        _tc_mid_body,
        grid=(GRID,),
        in_specs=[
            pl.BlockSpec((NC, RB, D), lambda i: (0, i, 0)),
            pl.BlockSpec((RB, D), lambda i: (i, 0)),
            pl.BlockSpec((RB, 1), lambda i: (i, 0)),
            pl.BlockSpec((1, D), lambda i: (0, 0)),
            pl.BlockSpec((D, D), lambda i: (0, 0)),
        ],
        out_specs=pl.BlockSpec((RB, D), lambda i: (i, 0)),
        out_shape=jax.ShapeDtypeStruct((NP, D), jnp.float32),
    )(aggp, g, dinv_col, b_row, w)


def _tc_final_body(aggp_ref, g_ref, dinv_ref, b_ref, out_ref):
    agg = aggp_ref[0] + aggp_ref[1]
    out_ref[...] = dinv_ref[...] * (agg + g_ref[...]) + b_ref[...]


def _tc_final(aggp, g, dinv_col, b_row):
    return pl.pallas_call(
        _tc_final_body,
        grid=(GRID,),
        in_specs=[
            pl.BlockSpec((NC, RB, D), lambda i: (0, i, 0)),
            pl.BlockSpec((RB, D), lambda i: (i, 0)),
            pl.BlockSpec((RB, 1), lambda i: (i, 0)),
            pl.BlockSpec((1, D), lambda i: (0, 0)),
        ],
        out_specs=pl.BlockSpec((RB, D), lambda i: (i, 0)),
        out_shape=jax.ShapeDtypeStruct((NP, D), jnp.float32),
    )(aggp, g, dinv_col, b_row)


@jax.jit
def kernel(x, edge_index, edge_attr, W1, b1, W2, b2):
    E = edge_index.shape[1]
    src = edge_index[0].astype(jnp.int32)
    dst = edge_index[1].astype(jnp.int32)
    ew = edge_attr[:, 0].astype(jnp.float32)

    src_p = jnp.pad(src, (0, EP - E)).reshape(NCHUNK, CH)
    dst_p = jnp.pad(dst, (0, EP - E)).reshape(NCHUNK, CH)
    ew_p = jnp.pad(ew, (0, EP - E)).reshape(NCHUNK, CH)
    x_p = jnp.pad(x, ((0, NP - N), (0, 0)))

    degp = _sc_deg(dst_p, ew_p)
    dinv_col = _tc_dinv(degp).reshape(NP, 1)

    g1 = _tc_mm_scale(x_p, W1, dinv_col)
    agg1 = _sc_agg(src_p, dst_p, ew_p, g1)
    g2 = _tc_mid(agg1, g1, dinv_col, b1.reshape(1, D), W2)
    agg2 = _sc_agg(src_p, dst_p, ew_p, g2)
    out = _tc_final(agg2, g2, dinv_col, b2.reshape(1, D))
    return out[:N]


# speedup vs baseline: 9.7705x; 1.0622x over previous
import functools

import jax
import jax.numpy as jnp
from jax import lax
from jax.experimental import pallas as pl
from jax.experimental.pallas import tpu as pltpu
from jax.experimental.pallas import tpu_sc as plsc

N = 10000
NP = 10240
D = 128
EP = 327680
CH = 128
NC, NS = 2, 16
NW = NC * NS
NCHUNK = EP // CH
CPW = NCHUNK // NW
RPT = NP // NS
LANES = 16

_mesh = plsc.VectorSubcoreMesh(
    core_axis_name="c", subcore_axis_name="s", num_cores=NC, num_subcores=NS
)


def _zeros16():
    return jnp.zeros((LANES,), jnp.float32)


def _sc_deg_body(dst_hbm, ew_hbm, out_hbm, dst_all, ew_all, buf_v, deg_sp,
                 sem_i, sem_s):
    cid = lax.axis_index("c")
    sid = lax.axis_index("s")
    wid = sid * NC + cid

    ld_d = pltpu.async_copy(dst_hbm.at[pl.ds(wid * CPW, CPW)], dst_all, sem_i)
    ld_e = pltpu.async_copy(ew_hbm.at[pl.ds(wid * CPW, CPW)], ew_all, sem_i)

    for i in range(CH // LANES):
        buf_v[pl.ds(i * LANES, LANES)] = _zeros16()
    for i in range(RPT // CH):
        pltpu.sync_copy(buf_v, deg_sp.at[pl.ds(sid * RPT + i * CH, CH)])
    ld_d.wait()
    ld_e.wait()
    plsc.subcore_barrier()

    K = 8

    def fire(c):
        pltpu.async_copy(ew_all.at[c], deg_sp.at[dst_all.at[c]], sem_s,
                         add=True)

    def drain(c):
        pltpu.make_async_copy(ew_all.at[c], deg_sp.at[dst_all.at[c]],
                              sem_s).wait()

    def body(i, carry):
        fire(K + i)
        drain(i)
        return carry

    for c in range(K):
        fire(c)
    lax.fori_loop(0, CPW - K, body, 0)
    for c in range(K):
        drain(CPW - K + c)
    plsc.subcore_barrier()

    def wr(i, carry):
        off = sid * RPT + i * CH
        pltpu.sync_copy(deg_sp.at[pl.ds(off, CH)], buf_v)
        pltpu.sync_copy(buf_v, out_hbm.at[cid, pl.ds(off, CH)])
        return carry

    lax.fori_loop(0, RPT // CH, wr, 0)


_sc_deg = pl.kernel(
    _sc_deg_body,
    out_type=jax.ShapeDtypeStruct((NC, NP), jnp.float32),
    mesh=_mesh,
    scratch_types=[
        pltpu.VMEM((CPW, CH), jnp.int32),
        pltpu.VMEM((CPW, CH), jnp.float32),
        pltpu.VMEM((CH,), jnp.float32),
        pltpu.VMEM_SHARED((NP,), jnp.float32),
        pltpu.SemaphoreType.DMA,
        pltpu.SemaphoreType.DMA,
    ],
)


PB = 8
FAST_CID = 0
CPW_F = 112
CPW_S = 48
NB_F = CPW_F // PB
NB_S = CPW_S // PB


def _sc_agg_body(src_hbm, dst_hbm, ew_hbm, g_hbm, out_hbm,
                 srcA, dstA, ewA, srcB, dstB, ewB, buf0, buf1, agg_sp,
                 sem_i, g0, g1, s0, s1):
    cid = lax.axis_index("c")
    sid = lax.axis_index("s")

    is_fast = cid == FAST_CID
    my_cpw = jnp.where(is_fast, CPW_F, CPW_S)
    nb = jnp.where(is_fast, NB_F, NB_S)
    start_w = jnp.where(is_fast, 0, NS * CPW_F) + sid * my_cpw

    def idx_refill(bidx, S, Dd, Ew):
        off = start_w + bidx * PB
        pltpu.async_copy(src_hbm.at[pl.ds(off, PB)], S, sem_i)
        pltpu.async_copy(dst_hbm.at[pl.ds(off, PB)], Dd, sem_i)
        pltpu.async_copy(ew_hbm.at[pl.ds(off, PB)], Ew, sem_i)

    def idx_wait(bidx, S, Dd, Ew):
        off = start_w + bidx * PB
        pltpu.make_async_copy(src_hbm.at[pl.ds(off, PB)], S, sem_i).wait()
        pltpu.make_async_copy(dst_hbm.at[pl.ds(off, PB)], Dd, sem_i).wait()
        pltpu.make_async_copy(ew_hbm.at[pl.ds(off, PB)], Ew, sem_i).wait()

    idx_refill(0, srcA, dstA, ewA)
    idx_refill(1, srcB, dstB, ewB)

    def zb(i, carry):
        for k in range(D // LANES):
            buf0[i, pl.ds(k * LANES, LANES)] = _zeros16()
        return carry

    lax.fori_loop(0, CH, zb, 0)

    def zs(i, carry):
        pltpu.sync_copy(buf0, agg_sp.at[pl.ds(sid * RPT + i * CH, CH)])
        return carry

    lax.fori_loop(0, RPT // CH, zs, 0)
    idx_wait(0, srcA, dstA, ewA)
    plsc.subcore_barrier()

    def start_gather(idx_ref, buf, sem):
        pltpu.async_copy(g_hbm.at[idx_ref], buf, sem)

    def wait_gather(idx_ref, buf, sem):
        pltpu.make_async_copy(g_hbm.at[idx_ref], buf, sem).wait()

    def start_scatter(idx_ref, buf, sem):
        pltpu.async_copy(buf, agg_sp.at[idx_ref], sem, add=True)

    def wait_scatter(idx_ref, buf, sem):
        pltpu.make_async_copy(buf, agg_sp.at[idx_ref], sem).wait()

    def scale(ew_row, buf):
        def grp(gi, carry):
            e0 = gi * LANES
            ewg = ew_row[pl.ds(e0, LANES)]
            for j in range(LANES):
                w = jnp.broadcast_to(ewg[j], (LANES,))
                for k in range(D // LANES):
                    sl = pl.ds(k * LANES, LANES)
                    buf[e0 + j, sl] = buf[e0 + j, sl] * w
            return carry

        lax.fori_loop(0, CH // LANES, grp, 0)

    start_gather(srcA.at[0], buf0, g0)
    start_gather(srcA.at[1], buf1, g1)

    def section(b, S, Dd, Ew, S2, D2, E2):
        def half(i, r, nxt, buf, gsem, ssem):
            wait_gather(S.at[r], buf, gsem)
            scale(Ew.at[r], buf)
            start_scatter(Dd.at[r], buf, ssem)
            wait_scatter(Dd.at[r], buf, ssem)

            @pl.when(i < PB // 2 - 1)
            def _():
                start_gather(S.at[r + 2], buf, gsem)

            @pl.when((i == PB // 2 - 1) & (b + 1 < nb))
            def _():
                if nxt == 0:
                    idx_wait(b + 1, S2, D2, E2)
                start_gather(S2.at[nxt], buf, gsem)

        def pair(i, carry):
            half(i, 2 * i, 0, buf0, g0, s0)
            half(i, 2 * i + 1, 1, buf1, g1, s1)
            return carry

        lax.fori_loop(0, PB // 2, pair, 0)

    def outer(ob, carry):
        bA = 2 * ob
        bB = 2 * ob + 1

        section(bA, srcA, dstA, ewA, srcB, dstB, ewB)

        @pl.when(bA + 2 < nb)
        def _():
            idx_refill(bA + 2, srcA, dstA, ewA)

        section(bB, srcB, dstB, ewB, srcA, dstA, ewA)

        @pl.when(bB + 2 < nb)
        def _():
            idx_refill(bB + 2, srcB, dstB, ewB)

        return carry

    lax.fori_loop(0, nb // 2, outer, 0)
    plsc.subcore_barrier()

    def wr(i, carry):
        off = sid * RPT + i * CH
        pltpu.sync_copy(agg_sp.at[pl.ds(off, CH)], buf0)
        pltpu.sync_copy(buf0, out_hbm.at[cid, pl.ds(off, CH)])
        return carry

    lax.fori_loop(0, RPT // CH, wr, 0)


_sc_agg = pl.kernel(
    _sc_agg_body,
    out_type=jax.ShapeDtypeStruct((NC, NP, D), jnp.float32),
    mesh=_mesh,
    scratch_types=[
        pltpu.VMEM((PB, CH), jnp.int32),
        pltpu.VMEM((PB, CH), jnp.int32),
        pltpu.VMEM((PB, CH), jnp.float32),
        pltpu.VMEM((PB, CH), jnp.int32),
        pltpu.VMEM((PB, CH), jnp.int32),
        pltpu.VMEM((PB, CH), jnp.float32),
        pltpu.VMEM((CH, D), jnp.float32),
        pltpu.VMEM((CH, D), jnp.float32),
        pltpu.VMEM_SHARED((NP, D), jnp.float32),
        pltpu.SemaphoreType.DMA,
        pltpu.SemaphoreType.DMA,
        pltpu.SemaphoreType.DMA,
        pltpu.SemaphoreType.DMA,
        pltpu.SemaphoreType.DMA,
    ],
)


def _tc_dinv_body(degp_ref, out_ref):
    deg = 1.0 + degp_ref[0] + degp_ref[1]
    out_ref[...] = lax.rsqrt(deg)


def _tc_dinv(degp):
    return pl.pallas_call(
        _tc_dinv_body,
        out_shape=jax.ShapeDtypeStruct((NP // D, D), jnp.float32),
    )(degp.reshape(NC, NP // D, D))


RB = 1024
GRID = NP // RB


def _tc_mm_scale_body(x_ref, w_ref, dinv_ref, out_ref):
    acc = jnp.dot(x_ref[...], w_ref[...], preferred_element_type=jnp.float32)
    out_ref[...] = acc * dinv_ref[...]


def _tc_mm_scale(x, w, dinv_col):
    return pl.pallas_call(
        _tc_mm_scale_body,
        grid=(GRID,),
        in_specs=[
            pl.BlockSpec((RB, D), lambda i: (i, 0)),
            pl.BlockSpec((D, D), lambda i: (0, 0)),
            pl.BlockSpec((RB, 1), lambda i: (i, 0)),
        ],
        out_specs=pl.BlockSpec((RB, D), lambda i: (i, 0)),
        out_shape=jax.ShapeDtypeStruct((NP, D), jnp.float32),
    )(x, w, dinv_col)


def _tc_mid_body(aggp_ref, g_ref, dinv_ref, b_ref, w_ref, out_ref):
    agg = aggp_ref[0] + aggp_ref[1]
    h = jnp.maximum(dinv_ref[...] * (agg + g_ref[...]) + b_ref[...], 0.0)
    acc = jnp.dot(h, w_ref[...], preferred_element_type=jnp.float32)
    out_ref[...] = acc * dinv_ref[...]


def _tc_mid(aggp, g, dinv_col, b_row, w):
    return pl.pallas_call(
        _tc_mid_body,
        grid=(GRID,),
        in_specs=[
            pl.BlockSpec((NC, RB, D), lambda i: (0, i, 0)),
            pl.BlockSpec((RB, D), lambda i: (i, 0)),
            pl.BlockSpec((RB, 1), lambda i: (i, 0)),
            pl.BlockSpec((1, D), lambda i: (0, 0)),
            pl.BlockSpec((D, D), lambda i: (0, 0)),
        ],
        out_specs=pl.BlockSpec((RB, D), lambda i: (i, 0)),
        out_shape=jax.ShapeDtypeStruct((NP, D), jnp.float32),
    )(aggp, g, dinv_col, b_row, w)


def _tc_final_body(aggp_ref, g_ref, dinv_ref, b_ref, out_ref):
    agg = aggp_ref[0] + aggp_ref[1]
    out_ref[...] = dinv_ref[...] * (agg + g_ref[...]) + b_ref[...]


def _tc_final(aggp, g, dinv_col, b_row):
    return pl.pallas_call(
        _tc_final_body,
        grid=(GRID,),
        in_specs=[
            pl.BlockSpec((NC, RB, D), lambda i: (0, i, 0)),
            pl.BlockSpec((RB, D), lambda i: (i, 0)),
            pl.BlockSpec((RB, 1), lambda i: (i, 0)),
            pl.BlockSpec((1, D), lambda i: (0, 0)),
        ],
        out_specs=pl.BlockSpec((RB, D), lambda i: (i, 0)),
        out_shape=jax.ShapeDtypeStruct((NP, D), jnp.float32),
    )(aggp, g, dinv_col, b_row)


@jax.jit
def kernel(x, edge_index, edge_attr, W1, b1, W2, b2):
    E = edge_index.shape[1]
    src = edge_index[0].astype(jnp.int32)
    dst = edge_index[1].astype(jnp.int32)
    ew = edge_attr[:, 0].astype(jnp.float32)

    src_p = jnp.pad(src, (0, EP - E)).reshape(NCHUNK, CH)
    dst_p = jnp.pad(dst, (0, EP - E)).reshape(NCHUNK, CH)
    ew_p = jnp.pad(ew, (0, EP - E)).reshape(NCHUNK, CH)
    x_p = jnp.pad(x, ((0, NP - N), (0, 0)))

    degp = _sc_deg(dst_p, ew_p)
    dinv_col = _tc_dinv(degp).reshape(NP, 1)

    g1 = _tc_mm_scale(x_p, W1, dinv_col)
    agg1 = _sc_agg(src_p, dst_p, ew_p, g1)
    g2 = _tc_mid(agg1, g1, dinv_col, b1.reshape(1, D), W2)
    agg2 = _sc_agg(src_p, dst_p, ew_p, g2)
    out = _tc_final(agg2, g2, dinv_col, b2.reshape(1, D))
    return out[:N]


# speedup vs baseline: 11.7245x; 1.2000x over previous
import functools

import jax
import jax.numpy as jnp
from jax import lax
from jax.experimental import pallas as pl
from jax.experimental.pallas import tpu as pltpu
from jax.experimental.pallas import tpu_sc as plsc

N = 10000
NP = 10240
D = 128
EP = 327680
CH = 128
NC, NS = 2, 16
NW = NC * NS
NCHUNK = EP // CH
CPW = NCHUNK // NW
RPT = NP // NS
LANES = 16

_mesh = plsc.VectorSubcoreMesh(
    core_axis_name="c", subcore_axis_name="s", num_cores=NC, num_subcores=NS
)


def _zeros16():
    return jnp.zeros((LANES,), jnp.float32)


def _sc_deg_body(dst_hbm, ew_hbm, out_hbm, dst_all, ew_all, buf_v, deg_sp,
                 sem_i, sem_s):
    cid = lax.axis_index("c")
    sid = lax.axis_index("s")
    wid = sid * NC + cid

    ld_d = pltpu.async_copy(dst_hbm.at[pl.ds(wid * CPW, CPW)], dst_all, sem_i)
    ld_e = pltpu.async_copy(ew_hbm.at[pl.ds(wid * CPW, CPW)], ew_all, sem_i)

    for i in range(CH // LANES):
        buf_v[pl.ds(i * LANES, LANES)] = _zeros16()
    for i in range(RPT // CH):
        pltpu.sync_copy(buf_v, deg_sp.at[pl.ds(sid * RPT + i * CH, CH)])
    ld_d.wait()
    ld_e.wait()
    plsc.subcore_barrier()

    K = 8

    def fire(c):
        pltpu.async_copy(ew_all.at[c], deg_sp.at[dst_all.at[c]], sem_s,
                         add=True)

    def drain(c):
        pltpu.make_async_copy(ew_all.at[c], deg_sp.at[dst_all.at[c]],
                              sem_s).wait()

    def body(i, carry):
        fire(K + i)
        drain(i)
        return carry

    for c in range(K):
        fire(c)
    lax.fori_loop(0, CPW - K, body, 0)
    for c in range(K):
        drain(CPW - K + c)
    plsc.subcore_barrier()

    def wr(i, carry):
        off = sid * RPT + i * CH
        pltpu.sync_copy(deg_sp.at[pl.ds(off, CH)], buf_v)
        pltpu.sync_copy(buf_v, out_hbm.at[cid, pl.ds(off, CH)])
        return carry

    lax.fori_loop(0, RPT // CH, wr, 0)


_sc_deg = pl.kernel(
    _sc_deg_body,
    out_type=jax.ShapeDtypeStruct((NC, NP), jnp.float32),
    mesh=_mesh,
    scratch_types=[
        pltpu.VMEM((CPW, CH), jnp.int32),
        pltpu.VMEM((CPW, CH), jnp.float32),
        pltpu.VMEM((CH,), jnp.float32),
        pltpu.VMEM_SHARED((NP,), jnp.float32),
        pltpu.SemaphoreType.DMA,
        pltpu.SemaphoreType.DMA,
    ],
)


PB = 8
FAST_CID = 0
CPW_F = 144
CPW_S = 16
NB_F = CPW_F // PB
NB_S = CPW_S // PB


def _sc_agg_body(src_hbm, dst_hbm, ew_hbm, g_hbm, out_hbm,
                 srcA, dstA, ewA, srcB, dstB, ewB, buf0, buf1, agg_sp,
                 sem_i, g0, g1, s0, s1):
    cid = lax.axis_index("c")
    sid = lax.axis_index("s")

    is_fast = cid == FAST_CID
    my_cpw = jnp.where(is_fast, CPW_F, CPW_S)
    nb = jnp.where(is_fast, NB_F, NB_S)
    start_w = jnp.where(is_fast, 0, NS * CPW_F) + sid * my_cpw

    def idx_refill(bidx, S, Dd, Ew):
        off = start_w + bidx * PB
        pltpu.async_copy(src_hbm.at[pl.ds(off, PB)], S, sem_i)
        pltpu.async_copy(dst_hbm.at[pl.ds(off, PB)], Dd, sem_i)
        pltpu.async_copy(ew_hbm.at[pl.ds(off, PB)], Ew, sem_i)

    def idx_wait(bidx, S, Dd, Ew):
        off = start_w + bidx * PB
        pltpu.make_async_copy(src_hbm.at[pl.ds(off, PB)], S, sem_i).wait()
        pltpu.make_async_copy(dst_hbm.at[pl.ds(off, PB)], Dd, sem_i).wait()
        pltpu.make_async_copy(ew_hbm.at[pl.ds(off, PB)], Ew, sem_i).wait()

    idx_refill(0, srcA, dstA, ewA)
    idx_refill(1, srcB, dstB, ewB)

    def zb(i, carry):
        for k in range(D // LANES):
            buf0[i, pl.ds(k * LANES, LANES)] = _zeros16()
        return carry

    lax.fori_loop(0, CH, zb, 0)

    def zs(i, carry):
        pltpu.sync_copy(buf0, agg_sp.at[pl.ds(sid * RPT + i * CH, CH)])
        return carry

    lax.fori_loop(0, RPT // CH, zs, 0)
    idx_wait(0, srcA, dstA, ewA)
    plsc.subcore_barrier()

    def start_gather(idx_ref, buf, sem):
        pltpu.async_copy(g_hbm.at[idx_ref], buf, sem)

    def wait_gather(idx_ref, buf, sem):
        pltpu.make_async_copy(g_hbm.at[idx_ref], buf, sem).wait()

    def start_scatter(idx_ref, buf, sem):
        pltpu.async_copy(buf, agg_sp.at[idx_ref], sem, add=True)

    def wait_scatter(idx_ref, buf, sem):
        pltpu.make_async_copy(buf, agg_sp.at[idx_ref], sem).wait()

    def scale(ew_row, buf):
        def grp(gi, carry):
            e0 = gi * LANES
            ewg = ew_row[pl.ds(e0, LANES)]
            for j in range(LANES):
                w = jnp.broadcast_to(ewg[j], (LANES,))
                for k in range(D // LANES):
                    sl = pl.ds(k * LANES, LANES)
                    buf[e0 + j, sl] = buf[e0 + j, sl] * w
            return carry

        lax.fori_loop(0, CH // LANES, grp, 0)

    start_gather(srcA.at[0], buf0, g0)
    start_gather(srcA.at[1], buf1, g1)

    def section(b, S, Dd, Ew, S2, D2, E2):
        def half(i, r, nxt, buf, gsem, ssem):
            wait_gather(S.at[r], buf, gsem)
            scale(Ew.at[r], buf)
            start_scatter(Dd.at[r], buf, ssem)
            wait_scatter(Dd.at[r], buf, ssem)

            @pl.when(i < PB // 2 - 1)
            def _():
                start_gather(S.at[r + 2], buf, gsem)

            @pl.when((i == PB // 2 - 1) & (b + 1 < nb))
            def _():
                if nxt == 0:
                    idx_wait(b + 1, S2, D2, E2)
                start_gather(S2.at[nxt], buf, gsem)

        def pair(i, carry):
            half(i, 2 * i, 0, buf0, g0, s0)
            half(i, 2 * i + 1, 1, buf1, g1, s1)
            return carry

        lax.fori_loop(0, PB // 2, pair, 0)

    def outer(ob, carry):
        bA = 2 * ob
        bB = 2 * ob + 1

        section(bA, srcA, dstA, ewA, srcB, dstB, ewB)

        @pl.when(bA + 2 < nb)
        def _():
            idx_refill(bA + 2, srcA, dstA, ewA)

        section(bB, srcB, dstB, ewB, srcA, dstA, ewA)

        @pl.when(bB + 2 < nb)
        def _():
            idx_refill(bB + 2, srcB, dstB, ewB)

        return carry

    lax.fori_loop(0, nb // 2, outer, 0)
    plsc.subcore_barrier()

    def wr(i, carry):
        off = sid * RPT + i * CH
        pltpu.sync_copy(agg_sp.at[pl.ds(off, CH)], buf0)
        pltpu.sync_copy(buf0, out_hbm.at[cid, pl.ds(off, CH)])
        return carry

    lax.fori_loop(0, RPT // CH, wr, 0)


_sc_agg = pl.kernel(
    _sc_agg_body,
    out_type=jax.ShapeDtypeStruct((NC, NP, D), jnp.float32),
    mesh=_mesh,
    scratch_types=[
        pltpu.VMEM((PB, CH), jnp.int32),
        pltpu.VMEM((PB, CH), jnp.int32),
        pltpu.VMEM((PB, CH), jnp.float32),
        pltpu.VMEM((PB, CH), jnp.int32),
        pltpu.VMEM((PB, CH), jnp.int32),
        pltpu.VMEM((PB, CH), jnp.float32),
        pltpu.VMEM((CH, D), jnp.float32),
        pltpu.VMEM((CH, D), jnp.float32),
        pltpu.VMEM_SHARED((NP, D), jnp.float32),
        pltpu.SemaphoreType.DMA,
        pltpu.SemaphoreType.DMA,
        pltpu.SemaphoreType.DMA,
        pltpu.SemaphoreType.DMA,
        pltpu.SemaphoreType.DMA,
    ],
)


def _tc_dinv_body(degp_ref, out_ref):
    deg = 1.0 + degp_ref[0] + degp_ref[1]
    out_ref[...] = lax.rsqrt(deg)


def _tc_dinv(degp):
    return pl.pallas_call(
        _tc_dinv_body,
        out_shape=jax.ShapeDtypeStruct((NP // D, D), jnp.float32),
    )(degp.reshape(NC, NP // D, D))


RB = 1024
GRID = NP // RB


def _tc_mm_scale_body(x_ref, w_ref, dinv_ref, out_ref):
    acc = jnp.dot(x_ref[...], w_ref[...], preferred_element_type=jnp.float32)
    out_ref[...] = acc * dinv_ref[...]


def _tc_mm_scale(x, w, dinv_col):
    return pl.pallas_call(
        _tc_mm_scale_body,
        grid=(GRID,),
        in_specs=[
            pl.BlockSpec((RB, D), lambda i: (i, 0)),
            pl.BlockSpec((D, D), lambda i: (0, 0)),
            pl.BlockSpec((RB, 1), lambda i: (i, 0)),
        ],
        out_specs=pl.BlockSpec((RB, D), lambda i: (i, 0)),
        out_shape=jax.ShapeDtypeStruct((NP, D), jnp.float32),
    )(x, w, dinv_col)


def _tc_mid_body(aggp_ref, g_ref, dinv_ref, b_ref, w_ref, out_ref):
    agg = aggp_ref[0] + aggp_ref[1]
    h = jnp.maximum(dinv_ref[...] * (agg + g_ref[...]) + b_ref[...], 0.0)
    acc = jnp.dot(h, w_ref[...], preferred_element_type=jnp.float32)
    out_ref[...] = acc * dinv_ref[...]


def _tc_mid(aggp, g, dinv_col, b_row, w):
    return pl.pallas_call(
        _tc_mid_body,
        grid=(GRID,),
        in_specs=[
            pl.BlockSpec((NC, RB, D), lambda i: (0, i, 0)),
            pl.BlockSpec((RB, D), lambda i: (i, 0)),
            pl.BlockSpec((RB, 1), lambda i: (i, 0)),
            pl.BlockSpec((1, D), lambda i: (0, 0)),
            pl.BlockSpec((D, D), lambda i: (0, 0)),
        ],
        out_specs=pl.BlockSpec((RB, D), lambda i: (i, 0)),
        out_shape=jax.ShapeDtypeStruct((NP, D), jnp.float32),
    )(aggp, g, dinv_col, b_row, w)


def _tc_final_body(aggp_ref, g_ref, dinv_ref, b_ref, out_ref):
    agg = aggp_ref[0] + aggp_ref[1]
    out_ref[...] = dinv_ref[...] * (agg + g_ref[...]) + b_ref[...]


def _tc_final(aggp, g, dinv_col, b_row):
    return pl.pallas_call(
        _tc_final_body,
        grid=(GRID,),
        in_specs=[
            pl.BlockSpec((NC, RB, D), lambda i: (0, i, 0)),
            pl.BlockSpec((RB, D), lambda i: (i, 0)),
            pl.BlockSpec((RB, 1), lambda i: (i, 0)),
            pl.BlockSpec((1, D), lambda i: (0, 0)),
        ],
        out_specs=pl.BlockSpec((RB, D), lambda i: (i, 0)),
        out_shape=jax.ShapeDtypeStruct((NP, D), jnp.float32),
    )(aggp, g, dinv_col, b_row)


@jax.jit
def kernel(x, edge_index, edge_attr, W1, b1, W2, b2):
    E = edge_index.shape[1]
    src = edge_index[0].astype(jnp.int32)
    dst = edge_index[1].astype(jnp.int32)
    ew = edge_attr[:, 0].astype(jnp.float32)

    src_p = jnp.pad(src, (0, EP - E)).reshape(NCHUNK, CH)
    dst_p = jnp.pad(dst, (0, EP - E)).reshape(NCHUNK, CH)
    ew_p = jnp.pad(ew, (0, EP - E)).reshape(NCHUNK, CH)
    x_p = jnp.pad(x, ((0, NP - N), (0, 0)))

    degp = _sc_deg(dst_p, ew_p)
    dinv_col = _tc_dinv(degp).reshape(NP, 1)

    g1 = _tc_mm_scale(x_p, W1, dinv_col)
    agg1 = _sc_agg(src_p, dst_p, ew_p, g1)
    g2 = _tc_mid(agg1, g1, dinv_col, b1.reshape(1, D), W2)
    agg2 = _sc_agg(src_p, dst_p, ew_p, g2)
    out = _tc_final(agg2, g2, dinv_col, b2.reshape(1, D))
    return out[:N]
